# baseline (speedup 1.0000x reference)
import functools

import jax
import jax.numpy as jnp
from jax import lax
from jax.experimental import pallas as pl
from jax.experimental.pallas import tpu as pltpu
from jax.experimental.pallas import tpu_sc as plsc

_EMB = 64
_NUM_CORES = 2
_NUM_SUBCORES = 16
_NUM_WORKERS = _NUM_CORES * _NUM_SUBCORES
_CHUNK = 256
_NBUF = 4


@functools.lru_cache(maxsize=None)
def _make_gather(n_rows: int):
    rows_per_w = n_rows // _NUM_WORKERS
    n_chunks = rows_per_w // _CHUNK
    n_groups = n_chunks // _NBUF
    mesh = plsc.VectorSubcoreMesh(core_axis_name="c", subcore_axis_name="s")

    @functools.partial(
        pl.kernel,
        out_type=jax.ShapeDtypeStruct((n_rows, _EMB), jnp.float32),
        mesh=mesh,
        compiler_params=pltpu.CompilerParams(use_tc_tiling_on_sc=False),
        scratch_types=[
            [pltpu.VMEM((_CHUNK,), jnp.int32) for _ in range(_NBUF)],
            [pltpu.VMEM((_CHUNK, _EMB), jnp.float32) for _ in range(_NBUF)],
            [pltpu.SemaphoreType.DMA for _ in range(_NBUF)],
            [pltpu.SemaphoreType.DMA for _ in range(_NBUF)],
        ],
    )
    def gather_kernel(ids_hbm, table_hbm, out_hbm, idx, rows, gsem, ssem):
        wid = lax.axis_index("s") * _NUM_CORES + lax.axis_index("c")
        base = wid * rows_per_w

        def body(j, carry):
            goff = base + j * _NBUF * _CHUNK
            for b in range(_NBUF):
                off = goff + b * _CHUNK

                @pl.when(j >= 1)
                def _(off=off, b=b):
                    prev = off - _NBUF * _CHUNK
                    pltpu.make_async_copy(
                        rows[b], out_hbm.at[pl.ds(prev, _CHUNK)], ssem[b]
                    ).wait()

                pltpu.sync_copy(ids_hbm.at[pl.ds(off, _CHUNK)], idx[b])
                pltpu.async_copy(table_hbm.at[idx[b]], rows[b], gsem[b])
            for b in range(_NBUF):
                off = goff + b * _CHUNK
                pltpu.make_async_copy(
                    table_hbm.at[idx[b]], rows[b], gsem[b]).wait()
                pltpu.async_copy(rows[b], out_hbm.at[pl.ds(off, _CHUNK)], ssem[b])
            return carry

        lax.fori_loop(0, n_groups, body, 0)

        for b in range(_NBUF):
            last = base + ((n_groups - 1) * _NBUF + b) * _CHUNK
            pltpu.make_async_copy(
                rows[b], out_hbm.at[pl.ds(last, _CHUNK)], ssem[b]).wait()

    return gather_kernel


def kernel(vocab_ids, table):
    b, s = vocab_ids.shape
    v, e = table.shape
    ids = lax.optimization_barrier(vocab_ids.reshape(-1).astype(jnp.int32))
    table_lin = lax.optimization_barrier(table.reshape(-1)).reshape(v, e)
    out = _make_gather(b * s)(ids, table_lin)
    return out.reshape(b, s, _EMB)
